# Initial kernel scaffold; baseline (speedup 1.0000x reference)
import jax
import jax.numpy as jnp
from jax.experimental import pallas as pl


def kernel(x_protein, x_ligand, edge_index_protein, edge_index_ligand, graph_ids_protein, graph_ids_ligand, Wp1, bp1, Wp2, bp2, Wl1, bl1, Wl2, bl2, Wl3, bl3, Wqkv, bqkv, Wproj, bproj, Wf1, bf1, Wf2, bf2, Wf3, bf3, Wout, bout):
    raise NotImplementedError("write your pallas kernel here")


# speedup vs baseline: 2.3894x; 2.3894x over previous
import functools

import numpy as np
import jax
import jax.numpy as jnp
from jax import lax
from jax.experimental import pallas as pl
from jax.experimental.pallas import tpu as pltpu
from jax.experimental.pallas import tpu_sc as plsc

NP_N = 10000
EP_E = 320000
NL_N = 5000
EL_E = 160000
GP_G = 100
GL_G = 49
D_IN = 128
K_HOP = 8
SEQ_L = 150
DIM_C = 45

NCORE = 2
NSUB = 16
CHUNK = 128
ZR = 8

NP_PAD = 10240
NL_PAD = 5120
EP_PAD = 323584
EL_PAD = 163840


def _sc_scatter_partials(tab, src, dst, n_pad, d, e_pad):
    nw = NCORE * NSUB
    chunks_per_tile = e_pad // (CHUNK * nw)
    rows_per_tile = n_pad // NSUB
    mesh = plsc.VectorSubcoreMesh(
        core_axis_name="c", subcore_axis_name="s",
        num_cores=NCORE, num_subcores=NSUB)

    @functools.partial(
        pl.kernel,
        out_type=jax.ShapeDtypeStruct((NCORE * n_pad, d), jnp.float32),
        mesh=mesh,
        compiler_params=pltpu.CompilerParams(use_tc_tiling_on_sc=False),
        scratch_types=[
            pltpu.VMEM((CHUNK,), jnp.int32),
            pltpu.VMEM((CHUNK,), jnp.int32),
            pltpu.VMEM((CHUNK, d), jnp.float32),
            pltpu.VMEM((ZR, d), jnp.float32),
            pltpu.VMEM_SHARED((n_pad, d), jnp.float32),
            pltpu.SemaphoreType.DMA,
        ],
    )
    def k(tab_hbm, src_hbm, dst_hbm, out_hbm,
          src_v, dst_v, rows_v, zero_v, acc_sh, sem):
        cid = lax.axis_index("c")
        sid = lax.axis_index("s")
        for i in range(ZR):
            for j in range(d // 16):
                zero_v[i, pl.ds(j * 16, 16)] = jnp.zeros((16,), jnp.float32)
        row0 = sid * rows_per_tile

        def zbody(i, carry):
            pltpu.sync_copy(zero_v, acc_sh.at[pl.ds(row0 + i * ZR, ZR)])
            return carry
        lax.fori_loop(0, rows_per_tile // ZR, zbody, 0)
        plsc.subcore_barrier()

        base_chunk = (cid * NSUB + sid) * chunks_per_tile

        def ebody(i, carry):
            e0 = (base_chunk + i) * CHUNK
            pltpu.sync_copy(src_hbm.at[pl.ds(e0, CHUNK)], src_v)
            pltpu.sync_copy(dst_hbm.at[pl.ds(e0, CHUNK)], dst_v)
            pltpu.async_copy(tab_hbm.at[src_v], rows_v, sem).wait()
            pltpu.sync_copy(rows_v, acc_sh.at[dst_v], add=True)
            return carry
        lax.fori_loop(0, chunks_per_tile, ebody, 0)
        plsc.subcore_barrier()

        out0 = cid * n_pad + row0
        pltpu.sync_copy(acc_sh.at[pl.ds(row0, rows_per_tile)],
                        out_hbm.at[pl.ds(out0, rows_per_tile)])

    return k(tab, src, dst)


def _tc_norm_from_deg(p):
    n_pad = p.shape[1]

    def body(p_ref, o_ref):
        deg = p_ref[0] + p_ref[1]
        o_ref[...] = lax.rsqrt(jnp.maximum(deg, 1.0))

    return pl.pallas_call(
        body, out_shape=jax.ShapeDtypeStruct((n_pad, 16), jnp.float32))(p)


def _tc_scale(x, norm16):
    n_pad, d = x.shape
    BN = 1024

    def body(x_ref, n_ref, o_ref):
        o_ref[...] = x_ref[...] * n_ref[...][:, :1]

    return pl.pallas_call(
        body, grid=(n_pad // BN,),
        in_specs=[pl.BlockSpec((BN, d), lambda i: (i, 0)),
                  pl.BlockSpec((BN, 16), lambda i: (i, 0))],
        out_specs=pl.BlockSpec((BN, d), lambda i: (i, 0)),
        out_shape=jax.ShapeDtypeStruct((n_pad, d), jnp.float32))(x, norm16)


def _tc_combine(p, norm16):
    _, n_pad, d = p.shape
    BN = 1024

    def body(p_ref, n_ref, h_ref, s_ref):
        nrm = n_ref[...][:, :1]
        h = (p_ref[0] + p_ref[1]) * nrm
        h_ref[...] = h
        s_ref[...] = h * nrm

    return pl.pallas_call(
        body, grid=(n_pad // BN,),
        in_specs=[pl.BlockSpec((2, BN, d), lambda i: (0, i, 0)),
                  pl.BlockSpec((BN, 16), lambda i: (i, 0))],
        out_specs=[pl.BlockSpec((BN, d), lambda i: (i, 0)),
                   pl.BlockSpec((BN, d), lambda i: (i, 0))],
        out_shape=[jax.ShapeDtypeStruct((n_pad, d), jnp.float32),
                   jax.ShapeDtypeStruct((n_pad, d), jnp.float32)])(p, norm16)


def _tc_tag_matmul(feats, w_pad, b_pad):
    k1, n_pad, d_in_p = feats.shape
    d_out_p = w_pad.shape[2]
    BN = 1024

    def body(f_ref, w_ref, b_ref, h_ref):
        acc = jnp.zeros((BN, d_out_p), jnp.float32)
        for k in range(k1):
            acc = acc + jnp.dot(f_ref[k], w_ref[k],
                                preferred_element_type=jnp.float32)
        y = jnp.maximum(acc + b_ref[...], 0.0)
        ss = jnp.sum(y * y, axis=1, keepdims=True)
        h_ref[...] = y / jnp.maximum(jnp.sqrt(ss), 1e-12)

    return pl.pallas_call(
        body, grid=(n_pad // BN,),
        in_specs=[pl.BlockSpec((k1, BN, d_in_p), lambda i: (0, i, 0)),
                  pl.BlockSpec((k1, d_in_p, d_out_p), lambda i: (0, 0, 0)),
                  pl.BlockSpec((1, d_out_p), lambda i: (0, 0))],
        out_specs=pl.BlockSpec((BN, d_out_p), lambda i: (i, 0)),
        out_shape=jax.ShapeDtypeStruct((n_pad, d_out_p), jnp.float32))(
            feats, w_pad, b_pad)


def _tc_segment_max(h, ids, n_seg):
    n_pad, d = h.shape
    g_pad = (n_seg + 7) // 8 * 8

    def body(h_ref, id_ref, o_ref):
        def gbody(g, carry):
            m = id_ref[...] == g
            vals = jnp.where(m, h_ref[...], -jnp.inf)
            mx = jnp.max(vals, axis=0, keepdims=True)
            o_ref[pl.ds(g, 1), :] = jnp.where(jnp.isfinite(mx), mx, 0.0)
            return carry
        lax.fori_loop(0, n_seg, gbody, 0)

    return pl.pallas_call(
        body, out_shape=jax.ShapeDtypeStruct((g_pad, d), jnp.float32))(h, ids)


def _tc_attention(seq, mask, wqkv, bqkv, wproj, bproj):
    def body(x_ref, m_ref, wq_ref, bq_ref, wp_ref, bp_ref, o_ref):
        x = x_ref[...]
        qkv = jnp.dot(x, wq_ref[...], preferred_element_type=jnp.float32)
        qkv = qkv + bq_ref[...]
        q = qkv[:, :DIM_C]
        kk = qkv[:, DIM_C:2 * DIM_C]
        v = qkv[:, 2 * DIM_C:]
        a = lax.dot_general(q, kk, (((1,), (1,)), ((), ())),
                            preferred_element_type=jnp.float32)
        a = a * (DIM_C ** -0.5)
        a = jnp.where(m_ref[...] == 0.0, -1e9, a)
        a = a - jnp.max(a, axis=1, keepdims=True)
        e = jnp.exp(a)
        p = e / jnp.sum(e, axis=1, keepdims=True)
        o = jnp.dot(p, v, preferred_element_type=jnp.float32)
        o_ref[...] = jnp.dot(o, wp_ref[...],
                             preferred_element_type=jnp.float32) + bp_ref[...]

    return pl.pallas_call(
        body, out_shape=jax.ShapeDtypeStruct((SEQ_L, DIM_C), jnp.float32))(
            seq, mask, wqkv, bqkv.reshape(1, -1), wproj, bproj.reshape(1, -1))


def _tc_mlp1(x, w, b):
    kp, n_out = w.shape
    BK = 512

    def body(x_ref, w_ref, b_ref, o_ref):
        @pl.when(pl.program_id(0) == 0)
        def _init():
            o_ref[...] = jnp.zeros_like(o_ref)

        o_ref[...] += jnp.dot(x_ref[...], w_ref[...],
                              preferred_element_type=jnp.float32)

        @pl.when(pl.program_id(0) == pl.num_programs(0) - 1)
        def _fin():
            o_ref[...] = jnp.maximum(o_ref[...] + b_ref[...], 0.0)

    return pl.pallas_call(
        body, grid=(kp // BK,),
        in_specs=[pl.BlockSpec((1, BK), lambda i: (0, i)),
                  pl.BlockSpec((BK, n_out), lambda i: (i, 0)),
                  pl.BlockSpec((1, n_out), lambda i: (0, 0))],
        out_specs=pl.BlockSpec((1, n_out), lambda i: (0, 0)),
        out_shape=jax.ShapeDtypeStruct((1, n_out), jnp.float32))(x, w, b)


def _tc_mlp_rest(x, w2, b2, w3, b3, w4, b4):
    def body(x_ref, w2_ref, b2_ref, w3_ref, b3_ref, w4_ref, b4_ref, o_ref):
        h = jnp.dot(x_ref[...], w2_ref[...], preferred_element_type=jnp.float32)
        h = jnp.maximum(h + b2_ref[...], 0.0)
        h = jnp.dot(h, w3_ref[...], preferred_element_type=jnp.float32)
        h = jnp.maximum(h + b3_ref[...], 0.0)
        z = jnp.dot(h, w4_ref[...], preferred_element_type=jnp.float32)
        z = z + b4_ref[...]
        o_ref[...] = 1.0 / (1.0 + jnp.exp(-z))

    return pl.pallas_call(
        body, out_shape=jax.ShapeDtypeStruct((1, w4.shape[1]), jnp.float32))(
            x, w2, b2, w3, b3, w4, b4)


def _attn_mask_np():
    n = GL_G + GP_G
    m = np.eye(SEQ_L, dtype=np.float32)
    m[n:, :] = 0.0
    m[:, n:] = 0.0
    m[:, n - 1] = 1.0
    m[n - 1, :] = 1.0
    m[n - 1, n - 1] = 0.0
    return m


def _pad2(x, r, c):
    out = jnp.zeros((r, c), jnp.float32)
    return out.at[:x.shape[0], :x.shape[1]].set(x)


def _run_graph(x, ei, n, n_pad, e, e_pad, layer_ws, layer_dims, gids, n_seg):
    src = jnp.concatenate(
        [ei[0], jnp.full((e_pad - e,), n_pad - 1, jnp.int32)])
    dst = jnp.concatenate(
        [ei[1], jnp.full((e_pad - e,), n_pad - 1, jnp.int32)])

    ones_tab = jnp.zeros((n_pad, 16), jnp.float32).at[:n].set(1.0)
    degp = _sc_scatter_partials(ones_tab, src, dst, n_pad, 16, e_pad)
    norm16 = _tc_norm_from_deg(degp.reshape(2, n_pad, 16))

    h = _pad2(x, n_pad, D_IN)
    for (w, b), (d_in, d_in_p, d_out, d_out_p) in zip(layer_ws, layer_dims):
        s = _tc_scale(h, norm16)
        feats = [h]
        for _hop in range(K_HOP):
            p = _sc_scatter_partials(s, src, dst, n_pad, d_in_p, e_pad)
            h_k, s = _tc_combine(p.reshape(2, n_pad, d_in_p), norm16)
            feats.append(h_k)
        fstack = jnp.stack(feats)
        w_r = w.reshape(K_HOP + 1, d_in, d_out)
        w_pad = jnp.zeros((K_HOP + 1, d_in_p, d_out_p), jnp.float32)
        w_pad = w_pad.at[:, :d_in, :d_out].set(w_r)
        b_pad = jnp.zeros((1, d_out_p), jnp.float32).at[0, :d_out].set(b)
        h = _tc_tag_matmul(fstack, w_pad, b_pad)

    ids = jnp.concatenate(
        [gids, jnp.full((n_pad - n,), np.int32(10 ** 6), jnp.int32)])
    rep = _tc_segment_max(h, ids.reshape(n_pad, 1), n_seg)
    return rep[:n_seg, :DIM_C]


def kernel(x_protein, x_ligand, edge_index_protein, edge_index_ligand,
           graph_ids_protein, graph_ids_ligand, Wp1, bp1, Wp2, bp2,
           Wl1, bl1, Wl2, bl2, Wl3, bl3, Wqkv, bqkv, Wproj, bproj,
           Wf1, bf1, Wf2, bf2, Wf3, bf3, Wout, bout):
    prot_dims = [(128, 128, 50, 64), (50, 64, 45, 48)]
    lig_dims = [(128, 128, 50, 64), (50, 64, 45, 48), (45, 48, 45, 48)]

    prot_rep = _run_graph(
        x_protein, edge_index_protein, NP_N, NP_PAD, EP_E, EP_PAD,
        [(Wp1, bp1), (Wp2, bp2)], prot_dims, graph_ids_protein, GP_G)
    lig_rep = _run_graph(
        x_ligand, edge_index_ligand, NL_N, NL_PAD, EL_E, EL_PAD,
        [(Wl1, bl1), (Wl2, bl2), (Wl3, bl3)], lig_dims,
        graph_ids_ligand, GL_G)

    seq = jnp.concatenate(
        [lig_rep, prot_rep,
         jnp.zeros((SEQ_L - GL_G - GP_G, DIM_C), jnp.float32)], axis=0)
    mask = jnp.asarray(_attn_mask_np())
    att = _tc_attention(seq, mask, Wqkv, bqkv, Wproj, bproj)

    xh = att.reshape(1, SEQ_L * DIM_C)
    xh_p = _pad2(xh, 1, 7168)
    w1 = _pad2(Wf1, 7168, 2048)
    b1 = _pad2(bf1.reshape(1, -1), 1, 2048)
    h1 = _tc_mlp1(xh_p, w1, b1)

    w2 = _pad2(Wf2, 2048, 1024)
    b2 = _pad2(bf2.reshape(1, -1), 1, 1024)
    w3 = _pad2(Wf3, 1024, 512)
    b3 = _pad2(bf3.reshape(1, -1), 1, 512)
    w4 = _pad2(Wout, 512, 128)
    b4 = _pad2(bout.reshape(1, -1), 1, 128)
    out = _tc_mlp_rest(h1, w2, b2, w3, b3, w4, b4)
    return out[0:1, 0:1]
